# baseline (speedup 1.0000x reference)
import functools

import jax
import jax.numpy as jnp
from jax import lax
from jax.experimental import pallas as pl
from jax.experimental.pallas import tpu as pltpu
from jax.experimental.pallas import tpu_sc as plsc

N = 10000
E = 160000
D = 256
NP = 10240
PADN = 10016
NS = 16
NC = 2
K = 80
EPT = 10240
EPAD = NS * EPT
NCHUNK = EPT // K
RPT = NP // NS
HD = D // NC

BLKR = 2048
NBLK = NP // BLKR

_mesh = plsc.VectorSubcoreMesh(core_axis_name="c", subcore_axis_name="s")


@functools.partial(
    pl.kernel,
    out_type=jax.ShapeDtypeStruct((NC * NS, NP), jnp.float32),
    mesh=_mesh,
    scratch_types=[
        pltpu.VMEM((NCHUNK, K), jnp.int32),
        pltpu.VMEM((NP,), jnp.float32),
        pltpu.VMEM((K,), jnp.float32),
        pltpu.VMEM_SHARED((NS * NP,), jnp.float32),
        pltpu.SemaphoreType.DMA,
    ],
)
def _deg_kernel(src_hbm, dst_hbm, out_hbm, idxv, zbuf, ones_v, hist_sh, sem):
    c = lax.axis_index("c")
    s = lax.axis_index("s")

    @pl.when(c == 0)
    def _():
        pltpu.sync_copy(src_hbm.at[s], idxv)

    @pl.when(c == 1)
    def _():
        pltpu.sync_copy(dst_hbm.at[s], idxv)

    one16 = jnp.ones((16,), jnp.float32)
    z16 = jnp.zeros((16,), jnp.float32)

    def _fill_ones(i, _):
        ones_v[pl.ds(i * 16, 16)] = one16
        return 0
    lax.fori_loop(0, K // 16, _fill_ones, 0)

    def _fill_zero(i, _):
        zbuf[pl.ds(i * 16, 16)] = z16
        return 0
    lax.fori_loop(0, NP // 16, _fill_zero, 0)
    pltpu.sync_copy(zbuf, hist_sh.at[pl.ds(s * NP, NP)])

    off = s * NP
    def _adj(i, _):
        for j in range(K // 16):
            idxv[i, pl.ds(j * 16, 16)] = idxv[i, pl.ds(j * 16, 16)] + off
        return 0
    lax.fori_loop(0, NCHUNK, _adj, 0)

    def _fire(i, _):
        pltpu.async_copy(ones_v, hist_sh.at[idxv.at[i]], sem, add=True)
        return 0
    lax.fori_loop(0, NCHUNK, _fire, 0)

    def _drain(i, _):
        pltpu.make_async_copy(ones_v, hist_sh.at[pl.ds(0, K)], sem).wait()
        return 0
    lax.fori_loop(0, NCHUNK, _drain, 0)

    pltpu.sync_copy(hist_sh.at[pl.ds(s * NP, NP)], out_hbm.at[c * NS + s])


@functools.partial(
    pl.kernel,
    out_type=jax.ShapeDtypeStruct((NC * NP, HD), jnp.float32),
    mesh=_mesh,
    scratch_types=[
        pltpu.VMEM((EPT,), jnp.int32),
        pltpu.VMEM((K,), jnp.int32),
        pltpu.VMEM((K,), jnp.int32),
        pltpu.VMEM((K, HD), jnp.float32),
        pltpu.VMEM((K, HD), jnp.float32),
        pltpu.VMEM_SHARED((NP, HD), jnp.float32),
        pltpu.SemaphoreType.DMA,
        pltpu.SemaphoreType.DMA,
        pltpu.SemaphoreType.DMA,
        pltpu.SemaphoreType.DMA,
    ],
)
def _edge_kernel(h_hbm, src_hbm, dst_hbm, out_hbm,
                 srcv, db0, db1, gb0, gb1, acc,
                 dl0, dl1, gs0, gs1):
    c = lax.axis_index("c")
    s = lax.axis_index("s")
    dbufs = (db0, db1)
    gbufs = (gb0, gb1)
    dlsems = (dl0, dl1)
    gsems = (gs0, gs1)

    off = c * NP
    ebase = s * EPT

    pltpu.sync_copy(src_hbm.at[pl.ds(ebase, EPT)], srcv)
    def _adj(i, _):
        srcv[pl.ds(i * 16, 16)] = srcv[pl.ds(i * 16, 16)] + off
        return 0
    lax.fori_loop(0, EPT // 16, _adj, 0)

    def _dl(i, j):
        pltpu.async_copy(dst_hbm.at[pl.ds(ebase + i * K, K)], dbufs[j], dlsems[j])

    def _dlw(j):
        pltpu.make_async_copy(dst_hbm.at[pl.ds(0, K)], dbufs[j], dlsems[j]).wait()

    def _gs(i, j):
        pltpu.async_copy(h_hbm.at[srcv.at[pl.ds(i * K, K)]], gbufs[j], gsems[j])

    def _gw(j):
        pltpu.make_async_copy(h_hbm.at[pl.ds(0, K)], gbufs[j], gsems[j]).wait()

    def _scat(j):
        pltpu.sync_copy(gbufs[j], acc.at[dbufs[j]], add=True)

    z16 = jnp.zeros((16,), jnp.float32)
    def _zrow(i, _):
        for q in range(HD // 16):
            gb0[i, pl.ds(q * 16, 16)] = z16
        return 0
    lax.fori_loop(0, K, _zrow, 0)
    for q in range(RPT // K):
        pltpu.sync_copy(gb0, acc.at[pl.ds(s * RPT + q * K, K)])
    plsc.subcore_barrier()

    _dl(0, 0)
    _dl(1, 1)
    _gs(0, 0)
    def _pair(k, _):
        i0 = 2 * k
        _gw(0); _gs(i0 + 1, 1); _dlw(0); _scat(0); _dl(i0 + 2, 0)
        _gw(1); _gs(i0 + 2, 0); _dlw(1); _scat(1); _dl(i0 + 3, 1)
        return 0
    lax.fori_loop(0, NCHUNK // 2 - 1, _pair, 0)
    _gw(0); _gs(NCHUNK - 1, 1); _dlw(0); _scat(0)
    _gw(1); _dlw(1); _scat(1)
    plsc.subcore_barrier()

    pltpu.sync_copy(acc.at[pl.ds(s * RPT, RPT)],
                    out_hbm.at[pl.ds(c * NP + s * RPT, RPT)])


def _degred_body(p_ref, rdo_ref, rdi_ref):
    p = p_ref[...]
    so = jnp.sum(p[:, :NS], axis=1, keepdims=True)
    si = jnp.sum(p[:, NS:], axis=1, keepdims=True)
    rdo_ref[...] = lax.rsqrt(jnp.maximum(so, 1.0))
    rdi_ref[...] = lax.rsqrt(jnp.maximum(si, 1.0))


def _mm_pre_body(x_ref, rdo_ref, w_ref, out_ref):
    xs = x_ref[...] * rdo_ref[...]
    h = jnp.dot(xs, w_ref[...], preferred_element_type=jnp.float32)
    out_ref[0] = h[:, :HD]
    out_ref[1] = h[:, HD:]


def _mm_mid_body(al_ref, ar_ref, rdi_ref, rdo_ref, b_ref, w_ref, out_ref):
    a = jnp.concatenate([al_ref[0], ar_ref[0]], axis=1)
    t = jnp.maximum(a * rdi_ref[...] + b_ref[...], 0.0)
    t = t * rdo_ref[...]
    h = jnp.dot(t, w_ref[...], preferred_element_type=jnp.float32)
    out_ref[0] = h[:, :HD]
    out_ref[1] = h[:, HD:]


def _fin_body(al_ref, ar_ref, rdi_ref, b_ref, out_ref):
    a = jnp.concatenate([al_ref[0], ar_ref[0]], axis=1)
    out_ref[...] = jnp.maximum(a * rdi_ref[...] + b_ref[...], 0.0)


def _degred(partials_t):
    return pl.pallas_call(
        _degred_body,
        grid=(NBLK,),
        in_specs=[
            pl.BlockSpec((BLKR, NC * NS), lambda r: (r, 0)),
        ],
        out_specs=[
            pl.BlockSpec((BLKR, 1), lambda r: (r, 0)),
            pl.BlockSpec((BLKR, 1), lambda r: (r, 0)),
        ],
        out_shape=[
            jax.ShapeDtypeStruct((NP, 1), jnp.float32),
            jax.ShapeDtypeStruct((NP, 1), jnp.float32),
        ],
    )(partials_t)


def _mm_pre(x_pad, rdo, W):
    return pl.pallas_call(
        _mm_pre_body,
        grid=(NBLK,),
        in_specs=[
            pl.BlockSpec((BLKR, D), lambda r: (r, 0)),
            pl.BlockSpec((BLKR, 1), lambda r: (r, 0)),
            pl.BlockSpec((D, D), lambda r: (0, 0)),
        ],
        out_specs=pl.BlockSpec((NC, BLKR, HD), lambda r: (0, r, 0)),
        out_shape=jax.ShapeDtypeStruct((NC, NP, HD), jnp.float32),
    )(x_pad, rdo, W)


def _mm_mid(agg, rdi, rdo, b, W):
    return pl.pallas_call(
        _mm_mid_body,
        grid=(NBLK,),
        in_specs=[
            pl.BlockSpec((1, BLKR, HD), lambda r: (0, r, 0)),
            pl.BlockSpec((1, BLKR, HD), lambda r: (1, r, 0)),
            pl.BlockSpec((BLKR, 1), lambda r: (r, 0)),
            pl.BlockSpec((BLKR, 1), lambda r: (r, 0)),
            pl.BlockSpec((1, D), lambda r: (0, 0)),
            pl.BlockSpec((D, D), lambda r: (0, 0)),
        ],
        out_specs=pl.BlockSpec((NC, BLKR, HD), lambda r: (0, r, 0)),
        out_shape=jax.ShapeDtypeStruct((NC, NP, HD), jnp.float32),
    )(agg, agg, rdi, rdo, b, W)


def _fin(agg, rdi, b):
    return pl.pallas_call(
        _fin_body,
        grid=(NBLK,),
        in_specs=[
            pl.BlockSpec((1, BLKR, HD), lambda r: (0, r, 0)),
            pl.BlockSpec((1, BLKR, HD), lambda r: (1, r, 0)),
            pl.BlockSpec((BLKR, 1), lambda r: (r, 0)),
            pl.BlockSpec((1, D), lambda r: (0, 0)),
        ],
        out_specs=pl.BlockSpec((BLKR, D), lambda r: (r, 0)),
        out_shape=jax.ShapeDtypeStruct((NP, D), jnp.float32),
    )(agg, agg, rdi, b)


def kernel(features, edge_index, W1, b1, W2, b2):
    pad = jnp.full((EPAD - E,), PADN, jnp.int32)
    src = jnp.concatenate([edge_index[0], pad])
    dst = jnp.concatenate([edge_index[1], pad])
    src3 = src.reshape(NS, NCHUNK, K)
    dst3 = dst.reshape(NS, NCHUNK, K)

    x_pad = jnp.pad(features, ((0, NP - N), (0, 0)))

    partials = _deg_kernel(src3, dst3)
    rdo, rdi = _degred(partials.T)

    b1r = b1.reshape(1, D)
    b2r = b2.reshape(1, D)

    h = _mm_pre(x_pad, rdo, W1)
    agg = _edge_kernel(h.reshape(NC * NP, HD), src, dst).reshape(NC, NP, HD)
    g = _mm_mid(agg, rdi, rdo, b1r, W2)
    agg2 = _edge_kernel(g.reshape(NC * NP, HD), src, dst).reshape(NC, NP, HD)
    out = _fin(agg2, rdi, b2r)
    return out[:N]


# speedup vs baseline: 1.0252x; 1.0252x over previous
import functools

import jax
import jax.numpy as jnp
from jax import lax
from jax.experimental import pallas as pl
from jax.experimental.pallas import tpu as pltpu
from jax.experimental.pallas import tpu_sc as plsc

N = 10000
E = 160000
D = 256
NP = 10240
PADN = 10016
NS = 16
NC = 2
K = 80
EPT = 10240
EPAD = NS * EPT
NCHUNK = EPT // K
RPT = NP // NS
HD = D // NC

BLKR = 2048
NBLK = NP // BLKR

_mesh = plsc.VectorSubcoreMesh(core_axis_name="c", subcore_axis_name="s")


@functools.partial(
    pl.kernel,
    out_type=jax.ShapeDtypeStruct((NC * NS, NP), jnp.float32),
    mesh=_mesh,
    scratch_types=[
        pltpu.VMEM((NCHUNK, K), jnp.int32),
        pltpu.VMEM((NP,), jnp.float32),
        pltpu.VMEM((K,), jnp.float32),
        pltpu.VMEM_SHARED((NS * NP,), jnp.float32),
        pltpu.SemaphoreType.DMA,
    ],
)
def _deg_kernel(src_hbm, dst_hbm, out_hbm, idxv, zbuf, ones_v, hist_sh, sem):
    c = lax.axis_index("c")
    s = lax.axis_index("s")

    @pl.when(c == 0)
    def _():
        pltpu.sync_copy(src_hbm.at[s], idxv)

    @pl.when(c == 1)
    def _():
        pltpu.sync_copy(dst_hbm.at[s], idxv)

    one16 = jnp.ones((16,), jnp.float32)
    z16 = jnp.zeros((16,), jnp.float32)

    def _fill_ones(i, _):
        ones_v[pl.ds(i * 16, 16)] = one16
        return 0
    lax.fori_loop(0, K // 16, _fill_ones, 0)

    def _fill_zero(i, _):
        zbuf[pl.ds(i * 16, 16)] = z16
        return 0
    lax.fori_loop(0, NP // 16, _fill_zero, 0)
    pltpu.sync_copy(zbuf, hist_sh.at[pl.ds(s * NP, NP)])

    off = s * NP
    def _adj(i, _):
        for j in range(K // 16):
            idxv[i, pl.ds(j * 16, 16)] = idxv[i, pl.ds(j * 16, 16)] + off
        return 0
    lax.fori_loop(0, NCHUNK, _adj, 0)

    def _fire(i, _):
        pltpu.async_copy(ones_v, hist_sh.at[idxv.at[i]], sem, add=True)
        return 0
    lax.fori_loop(0, NCHUNK, _fire, 0)

    def _drain(i, _):
        pltpu.make_async_copy(ones_v, hist_sh.at[pl.ds(0, K)], sem).wait()
        return 0
    lax.fori_loop(0, NCHUNK, _drain, 0)

    pltpu.sync_copy(hist_sh.at[pl.ds(s * NP, NP)], out_hbm.at[c * NS + s])


@functools.partial(
    pl.kernel,
    out_type=jax.ShapeDtypeStruct((NC * NP, HD), jnp.float32),
    mesh=_mesh,
    scratch_types=[
        pltpu.VMEM((EPT,), jnp.int32),
        pltpu.VMEM((NCHUNK, K), jnp.int32),
        pltpu.VMEM((K, HD), jnp.float32),
        pltpu.VMEM((K, HD), jnp.float32),
        pltpu.VMEM_SHARED((NP, HD), jnp.float32),
        pltpu.SemaphoreType.DMA,
        pltpu.SemaphoreType.DMA,
    ],
)
def _edge_kernel(h_hbm, src_hbm, dst_hbm, out_hbm,
                 srcv, dstv, gb0, gb1, acc, gs0, gs1):
    c = lax.axis_index("c")
    s = lax.axis_index("s")
    gbufs = (gb0, gb1)
    gsems = (gs0, gs1)

    off = c * NP
    ebase = s * EPT

    pltpu.sync_copy(src_hbm.at[pl.ds(ebase, EPT)], srcv)
    pltpu.sync_copy(dst_hbm.at[s], dstv)
    def _adj(i, _):
        srcv[pl.ds(i * 16, 16)] = srcv[pl.ds(i * 16, 16)] + off
        return 0
    lax.fori_loop(0, EPT // 16, _adj, 0)

    def _gs(i, j):
        pltpu.async_copy(h_hbm.at[srcv.at[pl.ds(i * K, K)]], gbufs[j], gsems[j])

    def _gw(j):
        pltpu.make_async_copy(h_hbm.at[pl.ds(0, K)], gbufs[j], gsems[j]).wait()

    def _scat(i, j):
        pltpu.sync_copy(gbufs[j], acc.at[dstv.at[i]], add=True)

    z16 = jnp.zeros((16,), jnp.float32)
    def _zrow(i, _):
        for q in range(HD // 16):
            gb0[i, pl.ds(q * 16, 16)] = z16
        return 0
    lax.fori_loop(0, K, _zrow, 0)
    for q in range(RPT // K):
        pltpu.sync_copy(gb0, acc.at[pl.ds(s * RPT + q * K, K)])
    plsc.subcore_barrier()

    _gs(0, 0)
    def _pair(k, _):
        i0 = 2 * k
        _gw(0); _gs(i0 + 1, 1); _scat(i0, 0)
        _gw(1); _gs(i0 + 2, 0); _scat(i0 + 1, 1)
        return 0
    lax.fori_loop(0, NCHUNK // 2 - 1, _pair, 0)
    _gw(0); _gs(NCHUNK - 1, 1); _scat(NCHUNK - 2, 0)
    _gw(1); _scat(NCHUNK - 1, 1)
    plsc.subcore_barrier()

    pltpu.sync_copy(acc.at[pl.ds(s * RPT, RPT)],
                    out_hbm.at[pl.ds(c * NP + s * RPT, RPT)])


def _degred_body(p_ref, rdo_ref, rdi_ref):
    p = p_ref[...]
    so = jnp.sum(p[:, :NS], axis=1, keepdims=True)
    si = jnp.sum(p[:, NS:], axis=1, keepdims=True)
    rdo_ref[...] = lax.rsqrt(jnp.maximum(so, 1.0))
    rdi_ref[...] = lax.rsqrt(jnp.maximum(si, 1.0))


def _mm_pre_body(x_ref, rdo_ref, w_ref, out_ref):
    xs = x_ref[...] * rdo_ref[...]
    h = jnp.dot(xs, w_ref[...], preferred_element_type=jnp.float32)
    out_ref[0] = h[:, :HD]
    out_ref[1] = h[:, HD:]


def _mm_mid_body(al_ref, ar_ref, rdi_ref, rdo_ref, b_ref, w_ref, out_ref):
    a = jnp.concatenate([al_ref[0], ar_ref[0]], axis=1)
    t = jnp.maximum(a * rdi_ref[...] + b_ref[...], 0.0)
    t = t * rdo_ref[...]
    h = jnp.dot(t, w_ref[...], preferred_element_type=jnp.float32)
    out_ref[0] = h[:, :HD]
    out_ref[1] = h[:, HD:]


def _fin_body(al_ref, ar_ref, rdi_ref, b_ref, out_ref):
    a = jnp.concatenate([al_ref[0], ar_ref[0]], axis=1)
    out_ref[...] = jnp.maximum(a * rdi_ref[...] + b_ref[...], 0.0)


def _degred(partials_t):
    return pl.pallas_call(
        _degred_body,
        grid=(NBLK,),
        in_specs=[
            pl.BlockSpec((BLKR, NC * NS), lambda r: (r, 0)),
        ],
        out_specs=[
            pl.BlockSpec((BLKR, 1), lambda r: (r, 0)),
            pl.BlockSpec((BLKR, 1), lambda r: (r, 0)),
        ],
        out_shape=[
            jax.ShapeDtypeStruct((NP, 1), jnp.float32),
            jax.ShapeDtypeStruct((NP, 1), jnp.float32),
        ],
    )(partials_t)


def _mm_pre(x_pad, rdo, W):
    return pl.pallas_call(
        _mm_pre_body,
        grid=(NBLK,),
        in_specs=[
            pl.BlockSpec((BLKR, D), lambda r: (r, 0)),
            pl.BlockSpec((BLKR, 1), lambda r: (r, 0)),
            pl.BlockSpec((D, D), lambda r: (0, 0)),
        ],
        out_specs=pl.BlockSpec((NC, BLKR, HD), lambda r: (0, r, 0)),
        out_shape=jax.ShapeDtypeStruct((NC, NP, HD), jnp.float32),
    )(x_pad, rdo, W)


def _mm_mid(agg, rdi, rdo, b, W):
    return pl.pallas_call(
        _mm_mid_body,
        grid=(NBLK,),
        in_specs=[
            pl.BlockSpec((1, BLKR, HD), lambda r: (0, r, 0)),
            pl.BlockSpec((1, BLKR, HD), lambda r: (1, r, 0)),
            pl.BlockSpec((BLKR, 1), lambda r: (r, 0)),
            pl.BlockSpec((BLKR, 1), lambda r: (r, 0)),
            pl.BlockSpec((1, D), lambda r: (0, 0)),
            pl.BlockSpec((D, D), lambda r: (0, 0)),
        ],
        out_specs=pl.BlockSpec((NC, BLKR, HD), lambda r: (0, r, 0)),
        out_shape=jax.ShapeDtypeStruct((NC, NP, HD), jnp.float32),
    )(agg, agg, rdi, rdo, b, W)


def _fin(agg, rdi, b):
    return pl.pallas_call(
        _fin_body,
        grid=(NBLK,),
        in_specs=[
            pl.BlockSpec((1, BLKR, HD), lambda r: (0, r, 0)),
            pl.BlockSpec((1, BLKR, HD), lambda r: (1, r, 0)),
            pl.BlockSpec((BLKR, 1), lambda r: (r, 0)),
            pl.BlockSpec((1, D), lambda r: (0, 0)),
        ],
        out_specs=pl.BlockSpec((BLKR, D), lambda r: (r, 0)),
        out_shape=jax.ShapeDtypeStruct((NP, D), jnp.float32),
    )(agg, agg, rdi, b)


def kernel(features, edge_index, W1, b1, W2, b2):
    pad = jnp.full((EPAD - E,), PADN, jnp.int32)
    src = jnp.concatenate([edge_index[0], pad])
    dst = jnp.concatenate([edge_index[1], pad])
    src3 = src.reshape(NS, NCHUNK, K)
    dst3 = dst.reshape(NS, NCHUNK, K)

    x_pad = jnp.pad(features, ((0, NP - N), (0, 0)))

    partials = _deg_kernel(src3, dst3)
    rdo, rdi = _degred(partials.T)

    b1r = b1.reshape(1, D)
    b2r = b2.reshape(1, D)

    h = _mm_pre(x_pad, rdo, W1)
    agg = _edge_kernel(h.reshape(NC * NP, HD), src, dst3).reshape(NC, NP, HD)
    g = _mm_mid(agg, rdi, rdo, b1r, W2)
    agg2 = _edge_kernel(g.reshape(NC * NP, HD), src, dst3).reshape(NC, NP, HD)
    out = _fin(agg2, rdi, b2r)
    return out[:N]


# speedup vs baseline: 1.8558x; 1.8102x over previous
import functools

import jax
import jax.numpy as jnp
from jax import lax
from jax.experimental import pallas as pl
from jax.experimental.pallas import tpu as pltpu
from jax.experimental.pallas import tpu_sc as plsc

N = 10000
E = 160000
D = 256
NP = 10240
PADN = 10016
NS = 16
NC = 2
K = 80
EPT = 10240
EPAD = NS * EPT
NCHUNK = EPT // K
RPT = NP // NS
HD = D // NC

BLKR = 2048
NBLK = NP // BLKR

_mesh = plsc.VectorSubcoreMesh(core_axis_name="c", subcore_axis_name="s")


@functools.partial(
    pl.kernel,
    out_type=jax.ShapeDtypeStruct((NC * NS, NP), jnp.float32),
    mesh=_mesh,
    scratch_types=[
        pltpu.VMEM((NCHUNK, K), jnp.int32),
        pltpu.VMEM((NP,), jnp.float32),
        pltpu.VMEM((K,), jnp.float32),
        pltpu.VMEM_SHARED((NS * NP,), jnp.float32),
        pltpu.SemaphoreType.DMA,
    ],
)
def _deg_kernel(src_hbm, dst_hbm, out_hbm, idxv, zbuf, ones_v, hist_sh, sem):
    c = lax.axis_index("c")
    s = lax.axis_index("s")

    @pl.when(c == 0)
    def _():
        pltpu.sync_copy(src_hbm.at[s], idxv)

    @pl.when(c == 1)
    def _():
        pltpu.sync_copy(dst_hbm.at[s], idxv)

    one16 = jnp.ones((16,), jnp.float32)
    z16 = jnp.zeros((16,), jnp.float32)

    def _fill_ones(i, _):
        ones_v[pl.ds(i * 16, 16)] = one16
        return 0
    lax.fori_loop(0, K // 16, _fill_ones, 0)

    def _fill_zero(i, _):
        zbuf[pl.ds(i * 16, 16)] = z16
        return 0
    lax.fori_loop(0, NP // 16, _fill_zero, 0)
    pltpu.sync_copy(zbuf, hist_sh.at[pl.ds(s * NP, NP)])

    off = s * NP
    def _adj(i, _):
        for j in range(K // 16):
            idxv[i, pl.ds(j * 16, 16)] = idxv[i, pl.ds(j * 16, 16)] + off
        return 0
    lax.fori_loop(0, NCHUNK, _adj, 0)

    def _fire(i, _):
        pltpu.async_copy(ones_v, hist_sh.at[idxv.at[i]], sem, add=True)
        return 0
    lax.fori_loop(0, NCHUNK, _fire, 0)

    def _drain(i, _):
        pltpu.make_async_copy(ones_v, hist_sh.at[pl.ds(0, K)], sem).wait()
        return 0
    lax.fori_loop(0, NCHUNK, _drain, 0)

    pltpu.sync_copy(hist_sh.at[pl.ds(s * NP, NP)], out_hbm.at[c * NS + s])


@functools.partial(
    pl.kernel,
    out_type=jax.ShapeDtypeStruct((NC * NP, HD), jnp.float32),
    mesh=_mesh,
    scratch_types=[
        pltpu.VMEM((EPT,), jnp.int32),
        pltpu.VMEM((NCHUNK, K), jnp.int32),
        pltpu.VMEM((K, HD), jnp.float32),
        pltpu.VMEM((K, HD), jnp.float32),
        pltpu.VMEM_SHARED((NP, HD), jnp.float32),
        pltpu.SemaphoreType.DMA,
        pltpu.SemaphoreType.DMA,
    ],
)
def _edge_kernel(h_hbm, src_hbm, dst_hbm, out_hbm,
                 srcv, dstv, gb0, gb1, acc, gs0, gs1):
    c = lax.axis_index("c")
    s = lax.axis_index("s")
    gbufs = (gb0, gb1)
    gsems = (gs0, gs1)

    off = c * NP
    ebase = s * EPT

    pltpu.sync_copy(src_hbm.at[pl.ds(ebase, EPT)], srcv)
    pltpu.sync_copy(dst_hbm.at[s], dstv)
    def _adj(i, _):
        srcv[pl.ds(i * 16, 16)] = srcv[pl.ds(i * 16, 16)] + off
        return 0
    lax.fori_loop(0, EPT // 16, _adj, 0)

    def _gs(i, j):
        pltpu.async_copy(h_hbm.at[srcv.at[pl.ds(i * K, K)]], gbufs[j], gsems[j])

    def _gw(j):
        pltpu.make_async_copy(h_hbm.at[pl.ds(0, K)], gbufs[j], gsems[j]).wait()

    def _scat(i, j):
        pltpu.sync_copy(gbufs[j], acc.at[dstv.at[i]], add=True)

    z16 = jnp.zeros((16,), jnp.float32)
    def _zrow(i, _):
        for q in range(HD // 16):
            gb0[i, pl.ds(q * 16, 16)] = z16
        return 0
    lax.fori_loop(0, K, _zrow, 0)
    for q in range(RPT // K):
        pltpu.sync_copy(gb0, acc.at[pl.ds(s * RPT + q * K, K)])
    plsc.subcore_barrier()

    _gs(0, 0)
    def _pair(k, _):
        i0 = 2 * k
        _gw(0); _gs(i0 + 1, 1); _scat(i0, 0)
        _gw(1); _gs(i0 + 2, 0); _scat(i0 + 1, 1)
        return 0
    lax.fori_loop(0, NCHUNK // 2 - 1, _pair, 0)
    _gw(0); _gs(NCHUNK - 1, 1); _scat(NCHUNK - 2, 0)
    _gw(1); _scat(NCHUNK - 1, 1)
    plsc.subcore_barrier()

    pltpu.sync_copy(acc.at[pl.ds(s * RPT, RPT)],
                    out_hbm.at[pl.ds(c * NP + s * RPT, RPT)])


def _degred_body(p_ref, rdo_ref, rdi_ref):
    p = p_ref[...]
    so = jnp.sum(p[:, :NS], axis=1, keepdims=True)
    si = jnp.sum(p[:, NS:], axis=1, keepdims=True)
    rdo_ref[...] = lax.rsqrt(jnp.maximum(so, 1.0))
    rdi_ref[...] = lax.rsqrt(jnp.maximum(si, 1.0))


def _mm_pre_body(x_ref, rdo_ref, w_ref, out_ref):
    xs = x_ref[...] * rdo_ref[...]
    h = jnp.dot(xs, w_ref[...], preferred_element_type=jnp.float32)
    out_ref[0] = h[:, :HD]
    out_ref[1] = h[:, HD:]


def _mm_mid_body(al_ref, ar_ref, rdi_ref, rdo_ref, b_ref, w_ref, out_ref):
    a = jnp.concatenate([al_ref[0], ar_ref[0]], axis=1)
    t = jnp.maximum(a * rdi_ref[...] + b_ref[...], 0.0)
    t = t * rdo_ref[...]
    h = jnp.dot(t, w_ref[...], preferred_element_type=jnp.float32)
    out_ref[0] = h[:, :HD]
    out_ref[1] = h[:, HD:]


def _fin_body(al_ref, ar_ref, rdi_ref, b_ref, out_ref):
    a = jnp.concatenate([al_ref[0], ar_ref[0]], axis=1)
    out_ref[...] = jnp.maximum(a * rdi_ref[...] + b_ref[...], 0.0)


def _degred(partials_t):
    return pl.pallas_call(
        _degred_body,
        grid=(NBLK,),
        in_specs=[
            pl.BlockSpec((BLKR, NC * NS), lambda r: (r, 0)),
        ],
        out_specs=[
            pl.BlockSpec((BLKR, 1), lambda r: (r, 0)),
            pl.BlockSpec((BLKR, 1), lambda r: (r, 0)),
        ],
        out_shape=[
            jax.ShapeDtypeStruct((NP, 1), jnp.float32),
            jax.ShapeDtypeStruct((NP, 1), jnp.float32),
        ],
    )(partials_t)


def _mm_pre(x_pad, rdo, W):
    return pl.pallas_call(
        _mm_pre_body,
        grid=(NBLK,),
        in_specs=[
            pl.BlockSpec((BLKR, D), lambda r: (r, 0)),
            pl.BlockSpec((BLKR, 1), lambda r: (r, 0)),
            pl.BlockSpec((D, D), lambda r: (0, 0)),
        ],
        out_specs=pl.BlockSpec((NC, BLKR, HD), lambda r: (0, r, 0)),
        out_shape=jax.ShapeDtypeStruct((NC, NP, HD), jnp.float32),
    )(x_pad, rdo, W)


def _mm_mid(agg, rdi, rdo, b, W):
    return pl.pallas_call(
        _mm_mid_body,
        grid=(NBLK,),
        in_specs=[
            pl.BlockSpec((1, BLKR, HD), lambda r: (0, r, 0)),
            pl.BlockSpec((1, BLKR, HD), lambda r: (1, r, 0)),
            pl.BlockSpec((BLKR, 1), lambda r: (r, 0)),
            pl.BlockSpec((BLKR, 1), lambda r: (r, 0)),
            pl.BlockSpec((1, D), lambda r: (0, 0)),
            pl.BlockSpec((D, D), lambda r: (0, 0)),
        ],
        out_specs=pl.BlockSpec((NC, BLKR, HD), lambda r: (0, r, 0)),
        out_shape=jax.ShapeDtypeStruct((NC, NP, HD), jnp.float32),
    )(agg, agg, rdi, rdo, b, W)


def _fin(agg, rdi, b):
    return pl.pallas_call(
        _fin_body,
        grid=(NBLK,),
        in_specs=[
            pl.BlockSpec((1, BLKR, HD), lambda r: (0, r, 0)),
            pl.BlockSpec((1, BLKR, HD), lambda r: (1, r, 0)),
            pl.BlockSpec((BLKR, 1), lambda r: (r, 0)),
            pl.BlockSpec((1, D), lambda r: (0, 0)),
        ],
        out_specs=pl.BlockSpec((BLKR, D), lambda r: (r, 0)),
        out_shape=jax.ShapeDtypeStruct((NP, D), jnp.float32),
    )(agg, agg, rdi, b)


def kernel(features, edge_index, W1, b1, W2, b2):
    pad = (jnp.arange(EPAD - E, dtype=jnp.int32) % (NP - N)) + N
    src = jnp.concatenate([edge_index[0], pad])
    dst = jnp.concatenate([edge_index[1], pad])
    src3 = src.reshape(NS, NCHUNK, K)
    dst3 = dst.reshape(NS, NCHUNK, K)

    x_pad = jnp.pad(features, ((0, NP - N), (0, 0)))

    partials = _deg_kernel(src3, dst3)
    rdo, rdi = _degred(partials.T)

    b1r = b1.reshape(1, D)
    b2r = b2.reshape(1, D)

    h = _mm_pre(x_pad, rdo, W1)
    agg = _edge_kernel(h.reshape(NC * NP, HD), src, dst3).reshape(NC, NP, HD)
    g = _mm_mid(agg, rdi, rdo, b1r, W2)
    agg2 = _edge_kernel(g.reshape(NC * NP, HD), src, dst3).reshape(NC, NP, HD)
    out = _fin(agg2, rdi, b2r)
    return out[:N]


# speedup vs baseline: 2.1680x; 1.1682x over previous
import functools

import jax
import jax.numpy as jnp
from jax import lax
from jax.experimental import pallas as pl
from jax.experimental.pallas import tpu as pltpu
from jax.experimental.pallas import tpu_sc as plsc

N = 10000
E = 160000
D = 256
NP = 10240
PADN = 10016
NS = 16
NC = 2
K = 128
EPT = 10240
EPAD = NS * EPT
NCHUNK = EPT // K
RPT = NP // NS
HD = D // NC

BLKR = 2048
NBLK = NP // BLKR

_mesh = plsc.VectorSubcoreMesh(core_axis_name="c", subcore_axis_name="s")


@functools.partial(
    pl.kernel,
    out_type=jax.ShapeDtypeStruct((NC * NS, NP), jnp.float32),
    mesh=_mesh,
    scratch_types=[
        pltpu.VMEM((NCHUNK, K), jnp.int32),
        pltpu.VMEM((NP,), jnp.float32),
        pltpu.VMEM((K,), jnp.float32),
        pltpu.VMEM_SHARED((NS * NP,), jnp.float32),
        pltpu.SemaphoreType.DMA,
    ],
)
def _deg_kernel(src_hbm, dst_hbm, out_hbm, idxv, zbuf, ones_v, hist_sh, sem):
    c = lax.axis_index("c")
    s = lax.axis_index("s")

    @pl.when(c == 0)
    def _():
        pltpu.sync_copy(src_hbm.at[s], idxv)

    @pl.when(c == 1)
    def _():
        pltpu.sync_copy(dst_hbm.at[s], idxv)

    one16 = jnp.ones((16,), jnp.float32)
    z16 = jnp.zeros((16,), jnp.float32)

    def _fill_ones(i, _):
        ones_v[pl.ds(i * 16, 16)] = one16
        return 0
    lax.fori_loop(0, K // 16, _fill_ones, 0)

    def _fill_zero(i, _):
        zbuf[pl.ds(i * 16, 16)] = z16
        return 0
    lax.fori_loop(0, NP // 16, _fill_zero, 0)
    pltpu.sync_copy(zbuf, hist_sh.at[pl.ds(s * NP, NP)])

    off = s * NP
    def _adj(i, _):
        for j in range(K // 16):
            idxv[i, pl.ds(j * 16, 16)] = idxv[i, pl.ds(j * 16, 16)] + off
        return 0
    lax.fori_loop(0, NCHUNK, _adj, 0)

    def _fire(i, _):
        pltpu.async_copy(ones_v, hist_sh.at[idxv.at[i]], sem, add=True)
        return 0
    lax.fori_loop(0, NCHUNK, _fire, 0)

    def _drain(i, _):
        pltpu.make_async_copy(ones_v, hist_sh.at[pl.ds(0, K)], sem).wait()
        return 0
    lax.fori_loop(0, NCHUNK, _drain, 0)

    pltpu.sync_copy(hist_sh.at[pl.ds(s * NP, NP)], out_hbm.at[c * NS + s])


@functools.partial(
    pl.kernel,
    out_type=jax.ShapeDtypeStruct((NC * NP, HD), jnp.float32),
    mesh=_mesh,
    scratch_types=[
        pltpu.VMEM((EPT,), jnp.int32),
        pltpu.VMEM((K,), jnp.int32),
        pltpu.VMEM((K,), jnp.int32),
        pltpu.VMEM((K, HD), jnp.float32),
        pltpu.VMEM((K, HD), jnp.float32),
        pltpu.VMEM_SHARED((NP, HD), jnp.float32),
        pltpu.SemaphoreType.DMA,
        pltpu.SemaphoreType.DMA,
        pltpu.SemaphoreType.DMA,
        pltpu.SemaphoreType.DMA,
    ],
)
def _edge_kernel(h_hbm, src_hbm, dst_hbm, out_hbm,
                 srcv, db0, db1, gb0, gb1, acc, dl0, dl1, gs0, gs1):
    c = lax.axis_index("c")
    s = lax.axis_index("s")
    gbufs = (gb0, gb1)
    gsems = (gs0, gs1)
    dbufs = (db0, db1)
    dlsems = (dl0, dl1)

    off = c * NP
    ebase = s * EPT

    pltpu.sync_copy(src_hbm.at[pl.ds(ebase, EPT)], srcv)
    def _adj(i, _):
        srcv[pl.ds(i * 16, 16)] = srcv[pl.ds(i * 16, 16)] + off
        return 0
    lax.fori_loop(0, EPT // 16, _adj, 0)

    def _dl(i, j):
        pltpu.async_copy(dst_hbm.at[pl.ds(ebase + i * K, K)], dbufs[j], dlsems[j])

    def _dlw(j):
        pltpu.make_async_copy(dst_hbm.at[pl.ds(0, K)], dbufs[j], dlsems[j]).wait()

    def _gs(i, j):
        pltpu.async_copy(h_hbm.at[srcv.at[pl.ds(i * K, K)]], gbufs[j], gsems[j])

    def _gw(j):
        pltpu.make_async_copy(h_hbm.at[pl.ds(0, K)], gbufs[j], gsems[j]).wait()

    def _scat(j):
        pltpu.sync_copy(gbufs[j], acc.at[dbufs[j]], add=True)

    z16 = jnp.zeros((16,), jnp.float32)
    def _zrow(i, _):
        for q in range(HD // 16):
            gb0[i, pl.ds(q * 16, 16)] = z16
        return 0
    lax.fori_loop(0, K, _zrow, 0)
    for q in range(RPT // K):
        pltpu.sync_copy(gb0, acc.at[pl.ds(s * RPT + q * K, K)])
    plsc.subcore_barrier()

    _dl(0, 0)
    _dl(1, 1)
    _gs(0, 0)
    def _pair(k, _):
        i0 = 2 * k
        _gw(0); _gs(i0 + 1, 1); _dlw(0); _scat(0); _dl(i0 + 2, 0)
        _gw(1); _gs(i0 + 2, 0); _dlw(1); _scat(1); _dl(i0 + 3, 1)
        return 0
    lax.fori_loop(0, NCHUNK // 2 - 1, _pair, 0)
    _gw(0); _gs(NCHUNK - 1, 1); _dlw(0); _scat(0)
    _gw(1); _dlw(1); _scat(1)
    plsc.subcore_barrier()

    pltpu.sync_copy(acc.at[pl.ds(s * RPT, RPT)],
                    out_hbm.at[pl.ds(c * NP + s * RPT, RPT)])


def _degred_body(p_ref, rdo_ref, rdi_ref):
    p = p_ref[...]
    so = jnp.sum(p[:, :NS], axis=1, keepdims=True)
    si = jnp.sum(p[:, NS:], axis=1, keepdims=True)
    rdo_ref[...] = lax.rsqrt(jnp.maximum(so, 1.0))
    rdi_ref[...] = lax.rsqrt(jnp.maximum(si, 1.0))


def _mm_pre_body(x_ref, rdo_ref, w_ref, out_ref):
    xs = x_ref[...] * rdo_ref[...]
    h = jnp.dot(xs, w_ref[...], preferred_element_type=jnp.float32)
    out_ref[0] = h[:, :HD]
    out_ref[1] = h[:, HD:]


def _mm_mid_body(al_ref, ar_ref, rdi_ref, rdo_ref, b_ref, w_ref, out_ref):
    a = jnp.concatenate([al_ref[0], ar_ref[0]], axis=1)
    t = jnp.maximum(a * rdi_ref[...] + b_ref[...], 0.0)
    t = t * rdo_ref[...]
    h = jnp.dot(t, w_ref[...], preferred_element_type=jnp.float32)
    out_ref[0] = h[:, :HD]
    out_ref[1] = h[:, HD:]


def _fin_body(al_ref, ar_ref, rdi_ref, b_ref, out_ref):
    a = jnp.concatenate([al_ref[0], ar_ref[0]], axis=1)
    out_ref[...] = jnp.maximum(a * rdi_ref[...] + b_ref[...], 0.0)


def _degred(partials_t):
    return pl.pallas_call(
        _degred_body,
        grid=(NBLK,),
        in_specs=[
            pl.BlockSpec((BLKR, NC * NS), lambda r: (r, 0)),
        ],
        out_specs=[
            pl.BlockSpec((BLKR, 1), lambda r: (r, 0)),
            pl.BlockSpec((BLKR, 1), lambda r: (r, 0)),
        ],
        out_shape=[
            jax.ShapeDtypeStruct((NP, 1), jnp.float32),
            jax.ShapeDtypeStruct((NP, 1), jnp.float32),
        ],
    )(partials_t)


def _mm_pre(x_pad, rdo, W):
    return pl.pallas_call(
        _mm_pre_body,
        grid=(NBLK,),
        in_specs=[
            pl.BlockSpec((BLKR, D), lambda r: (r, 0)),
            pl.BlockSpec((BLKR, 1), lambda r: (r, 0)),
            pl.BlockSpec((D, D), lambda r: (0, 0)),
        ],
        out_specs=pl.BlockSpec((NC, BLKR, HD), lambda r: (0, r, 0)),
        out_shape=jax.ShapeDtypeStruct((NC, NP, HD), jnp.float32),
    )(x_pad, rdo, W)


def _mm_mid(agg, rdi, rdo, b, W):
    return pl.pallas_call(
        _mm_mid_body,
        grid=(NBLK,),
        in_specs=[
            pl.BlockSpec((1, BLKR, HD), lambda r: (0, r, 0)),
            pl.BlockSpec((1, BLKR, HD), lambda r: (1, r, 0)),
            pl.BlockSpec((BLKR, 1), lambda r: (r, 0)),
            pl.BlockSpec((BLKR, 1), lambda r: (r, 0)),
            pl.BlockSpec((1, D), lambda r: (0, 0)),
            pl.BlockSpec((D, D), lambda r: (0, 0)),
        ],
        out_specs=pl.BlockSpec((NC, BLKR, HD), lambda r: (0, r, 0)),
        out_shape=jax.ShapeDtypeStruct((NC, NP, HD), jnp.float32),
    )(agg, agg, rdi, rdo, b, W)


def _fin(agg, rdi, b):
    return pl.pallas_call(
        _fin_body,
        grid=(NBLK,),
        in_specs=[
            pl.BlockSpec((1, BLKR, HD), lambda r: (0, r, 0)),
            pl.BlockSpec((1, BLKR, HD), lambda r: (1, r, 0)),
            pl.BlockSpec((BLKR, 1), lambda r: (r, 0)),
            pl.BlockSpec((1, D), lambda r: (0, 0)),
        ],
        out_specs=pl.BlockSpec((BLKR, D), lambda r: (r, 0)),
        out_shape=jax.ShapeDtypeStruct((NP, D), jnp.float32),
    )(agg, agg, rdi, b)


def kernel(features, edge_index, W1, b1, W2, b2):
    pad = (jnp.arange(EPAD - E, dtype=jnp.int32) % (NP - N)) + N
    src = jnp.concatenate([edge_index[0], pad])
    dst = jnp.concatenate([edge_index[1], pad])
    src3 = src.reshape(NS, NCHUNK, K)
    dst3 = dst.reshape(NS, NCHUNK, K)

    x_pad = jnp.pad(features, ((0, NP - N), (0, 0)))

    partials = _deg_kernel(src3, dst3)
    rdo, rdi = _degred(partials.T)

    b1r = b1.reshape(1, D)
    b2r = b2.reshape(1, D)

    h = _mm_pre(x_pad, rdo, W1)
    agg = _edge_kernel(h.reshape(NC * NP, HD), src, dst).reshape(NC, NP, HD)
    g = _mm_mid(agg, rdi, rdo, b1r, W2)
    agg2 = _edge_kernel(g.reshape(NC * NP, HD), src, dst).reshape(NC, NP, HD)
    out = _fin(agg2, rdi, b2r)
    return out[:N]


# speedup vs baseline: 2.2180x; 1.0231x over previous
import functools

import jax
import jax.numpy as jnp
from jax import lax
from jax.experimental import pallas as pl
from jax.experimental.pallas import tpu as pltpu
from jax.experimental.pallas import tpu_sc as plsc

N = 10000
E = 160000
D = 256
NP = 10240
PADN = 10016
NS = 16
NC = 2
K = 128
EPT = 10240
EPAD = NS * EPT
NCHUNK = EPT // K
RPT = NP // NS
HD = D // NC

BLKR = 2048
NBLK = NP // BLKR

_mesh = plsc.VectorSubcoreMesh(core_axis_name="c", subcore_axis_name="s")


@functools.partial(
    pl.kernel,
    out_type=jax.ShapeDtypeStruct((NC * NS, NP), jnp.float32),
    mesh=_mesh,
    scratch_types=[
        pltpu.VMEM((NCHUNK, K), jnp.int32),
        pltpu.VMEM((NP,), jnp.float32),
        pltpu.VMEM((K,), jnp.float32),
        pltpu.VMEM_SHARED((NS * NP,), jnp.float32),
        pltpu.SemaphoreType.DMA,
    ],
)
def _deg_kernel(src_hbm, dst_hbm, out_hbm, idxv, zbuf, ones_v, hist_sh, sem):
    c = lax.axis_index("c")
    s = lax.axis_index("s")

    @pl.when(c == 0)
    def _():
        pltpu.sync_copy(src_hbm.at[s], idxv)

    @pl.when(c == 1)
    def _():
        pltpu.sync_copy(dst_hbm.at[s], idxv)

    one16 = jnp.ones((16,), jnp.float32)
    z16 = jnp.zeros((16,), jnp.float32)

    def _fill_ones(i, _):
        ones_v[pl.ds(i * 16, 16)] = one16
        return 0
    lax.fori_loop(0, K // 16, _fill_ones, 0)

    def _fill_zero(i, _):
        zbuf[pl.ds(i * 16, 16)] = z16
        return 0
    lax.fori_loop(0, NP // 16, _fill_zero, 0)
    pltpu.sync_copy(zbuf, hist_sh.at[pl.ds(s * NP, NP)])

    off = s * NP
    def _adj(i, _):
        for j in range(K // 16):
            idxv[i, pl.ds(j * 16, 16)] = idxv[i, pl.ds(j * 16, 16)] + off
        return 0
    lax.fori_loop(0, NCHUNK, _adj, 0)

    def _fire(i, _):
        pltpu.async_copy(ones_v, hist_sh.at[idxv.at[i]], sem, add=True)
        return 0
    lax.fori_loop(0, NCHUNK, _fire, 0)

    def _drain(i, _):
        pltpu.make_async_copy(ones_v, hist_sh.at[pl.ds(0, K)], sem).wait()
        return 0
    lax.fori_loop(0, NCHUNK, _drain, 0)

    pltpu.sync_copy(hist_sh.at[pl.ds(s * NP, NP)], out_hbm.at[c * NS + s])


@functools.partial(
    pl.kernel,
    out_type=jax.ShapeDtypeStruct((NC * NP, HD), jnp.float32),
    mesh=_mesh,
    scratch_types=[
        pltpu.VMEM((EPT,), jnp.int32),
        pltpu.VMEM((K,), jnp.int32),
        pltpu.VMEM((K,), jnp.int32),
        pltpu.VMEM((K, HD), jnp.float32),
        pltpu.VMEM((K, HD), jnp.float32),
        pltpu.VMEM_SHARED((NP, HD), jnp.float32),
        pltpu.SemaphoreType.DMA,
        pltpu.SemaphoreType.DMA,
        pltpu.SemaphoreType.DMA,
        pltpu.SemaphoreType.DMA,
    ],
)
def _edge_kernel(h_hbm, src_hbm, dst_hbm, out_hbm,
                 srcv, db0, db1, gb0, gb1, acc, dl0, dl1, gs0, gs1):
    c = lax.axis_index("c")
    s = lax.axis_index("s")
    gbufs = (gb0, gb1)
    gsems = (gs0, gs1)
    dbufs = (db0, db1)
    dlsems = (dl0, dl1)

    off = c * NP
    ebase = s * EPT

    pltpu.sync_copy(src_hbm.at[pl.ds(ebase, EPT)], srcv)
    def _adj(i, _):
        srcv[pl.ds(i * 16, 16)] = srcv[pl.ds(i * 16, 16)] + off
        return 0
    lax.fori_loop(0, EPT // 16, _adj, 0)

    def _dl(i, j):
        pltpu.async_copy(dst_hbm.at[pl.ds(ebase + i * K, K)], dbufs[j], dlsems[j])

    def _dlw(j):
        pltpu.make_async_copy(dst_hbm.at[pl.ds(0, K)], dbufs[j], dlsems[j]).wait()

    def _gs(i, j):
        pltpu.async_copy(h_hbm.at[srcv.at[pl.ds(i * K, K)]], gbufs[j], gsems[j])

    def _gw(j):
        pltpu.make_async_copy(h_hbm.at[pl.ds(0, K)], gbufs[j], gsems[j]).wait()

    def _scat(j):
        pltpu.sync_copy(gbufs[j], acc.at[dbufs[j]], add=True)

    z16 = jnp.zeros((16,), jnp.float32)
    def _zrow(i, _):
        for q in range(HD // 16):
            gb0[i, pl.ds(q * 16, 16)] = z16
        return 0
    lax.fori_loop(0, K, _zrow, 0)
    for q in range(RPT // K):
        pltpu.sync_copy(gb0, acc.at[pl.ds(s * RPT + q * K, K)])
    plsc.subcore_barrier()

    _dl(0, 0)
    _dl(1, 1)
    _gs(0, 0)
    def _pair(k, _):
        i0 = 2 * k
        _gw(0); _gs(i0 + 1, 1); _dlw(0); _scat(0); _dl(i0 + 2, 0)
        _gw(1); _gs(i0 + 2, 0); _dlw(1); _scat(1); _dl(i0 + 3, 1)
        return 0
    lax.fori_loop(0, NCHUNK // 2 - 1, _pair, 0)
    _gw(0); _gs(NCHUNK - 1, 1); _dlw(0); _scat(0)
    _gw(1); _dlw(1); _scat(1)
    plsc.subcore_barrier()

    pltpu.sync_copy(acc.at[pl.ds(s * RPT, RPT)],
                    out_hbm.at[pl.ds(c * NP + s * RPT, RPT)])


def _mm_pre_body(x_ref, p_ref, w_ref, out_ref, rdo_ref, rdi_ref):
    p = p_ref[...]
    so = jnp.sum(p[:, :NS], axis=1, keepdims=True)
    si = jnp.sum(p[:, NS:], axis=1, keepdims=True)
    rdo = lax.rsqrt(jnp.maximum(so, 1.0))
    rdi = lax.rsqrt(jnp.maximum(si, 1.0))
    rdo_ref[...] = rdo
    rdi_ref[...] = rdi
    xs = x_ref[...] * rdo
    h = jnp.dot(xs, w_ref[...], preferred_element_type=jnp.float32)
    out_ref[0] = h[:, :HD]
    out_ref[1] = h[:, HD:]


def _mm_mid_body(al_ref, ar_ref, rdi_ref, rdo_ref, b_ref, w_ref, out_ref):
    a = jnp.concatenate([al_ref[0], ar_ref[0]], axis=1)
    t = jnp.maximum(a * rdi_ref[...] + b_ref[...], 0.0)
    t = t * rdo_ref[...]
    h = jnp.dot(t, w_ref[...], preferred_element_type=jnp.float32)
    out_ref[0] = h[:, :HD]
    out_ref[1] = h[:, HD:]


def _fin_body(al_ref, ar_ref, rdi_ref, b_ref, out_ref):
    a = jnp.concatenate([al_ref[0], ar_ref[0]], axis=1)
    out_ref[...] = jnp.maximum(a * rdi_ref[...] + b_ref[...], 0.0)


def _mm_pre(x_pad, partials_t, W):
    return pl.pallas_call(
        _mm_pre_body,
        grid=(NBLK,),
        in_specs=[
            pl.BlockSpec((BLKR, D), lambda r: (r, 0)),
            pl.BlockSpec((BLKR, NC * NS), lambda r: (r, 0)),
            pl.BlockSpec((D, D), lambda r: (0, 0)),
        ],
        out_specs=[
            pl.BlockSpec((NC, BLKR, HD), lambda r: (0, r, 0)),
            pl.BlockSpec((BLKR, 1), lambda r: (r, 0)),
            pl.BlockSpec((BLKR, 1), lambda r: (r, 0)),
        ],
        out_shape=[
            jax.ShapeDtypeStruct((NC, NP, HD), jnp.float32),
            jax.ShapeDtypeStruct((NP, 1), jnp.float32),
            jax.ShapeDtypeStruct((NP, 1), jnp.float32),
        ],
    )(x_pad, partials_t, W)


def _mm_mid(agg, rdi, rdo, b, W):
    return pl.pallas_call(
        _mm_mid_body,
        grid=(NBLK,),
        in_specs=[
            pl.BlockSpec((1, BLKR, HD), lambda r: (0, r, 0)),
            pl.BlockSpec((1, BLKR, HD), lambda r: (1, r, 0)),
            pl.BlockSpec((BLKR, 1), lambda r: (r, 0)),
            pl.BlockSpec((BLKR, 1), lambda r: (r, 0)),
            pl.BlockSpec((1, D), lambda r: (0, 0)),
            pl.BlockSpec((D, D), lambda r: (0, 0)),
        ],
        out_specs=pl.BlockSpec((NC, BLKR, HD), lambda r: (0, r, 0)),
        out_shape=jax.ShapeDtypeStruct((NC, NP, HD), jnp.float32),
    )(agg, agg, rdi, rdo, b, W)


def _fin(agg, rdi, b):
    return pl.pallas_call(
        _fin_body,
        grid=(NBLK,),
        in_specs=[
            pl.BlockSpec((1, BLKR, HD), lambda r: (0, r, 0)),
            pl.BlockSpec((1, BLKR, HD), lambda r: (1, r, 0)),
            pl.BlockSpec((BLKR, 1), lambda r: (r, 0)),
            pl.BlockSpec((1, D), lambda r: (0, 0)),
        ],
        out_specs=pl.BlockSpec((BLKR, D), lambda r: (r, 0)),
        out_shape=jax.ShapeDtypeStruct((NP, D), jnp.float32),
    )(agg, agg, rdi, b)


def kernel(features, edge_index, W1, b1, W2, b2):
    pad = (jnp.arange(EPAD - E, dtype=jnp.int32) % (NP - N)) + N
    src = jnp.concatenate([edge_index[0], pad])
    dst = jnp.concatenate([edge_index[1], pad])
    src3 = src.reshape(NS, NCHUNK, K)
    dst3 = dst.reshape(NS, NCHUNK, K)

    x_pad = jnp.pad(features, ((0, NP - N), (0, 0)))

    partials = _deg_kernel(src3, dst3)

    b1r = b1.reshape(1, D)
    b2r = b2.reshape(1, D)

    h, rdo, rdi = _mm_pre(x_pad, partials.T, W1)
    agg = _edge_kernel(h.reshape(NC * NP, HD), src, dst).reshape(NC, NP, HD)
    g = _mm_mid(agg, rdi, rdo, b1r, W2)
    agg2 = _edge_kernel(g.reshape(NC * NP, HD), src, dst).reshape(NC, NP, HD)
    out = _fin(agg2, rdi, b2r)
    return out[:N]


# speedup vs baseline: 2.2199x; 1.0009x over previous
import functools

import jax
import jax.numpy as jnp
from jax import lax
from jax.experimental import pallas as pl
from jax.experimental.pallas import tpu as pltpu
from jax.experimental.pallas import tpu_sc as plsc

N = 10000
E = 160000
D = 256
NP = 10240
PADN = 10016
NS = 16
NC = 2
K = 128
EPT = 10240
EPAD = NS * EPT
NCHUNK = EPT // K
RPT = NP // NS
HD = D // NC

BLKR = 2048
NBLK = NP // BLKR

_mesh = plsc.VectorSubcoreMesh(core_axis_name="c", subcore_axis_name="s")


@functools.partial(
    pl.kernel,
    out_type=jax.ShapeDtypeStruct((NC * NS, NP), jnp.float32),
    mesh=_mesh,
    scratch_types=[
        pltpu.VMEM((NCHUNK, K), jnp.int32),
        pltpu.VMEM((NP,), jnp.float32),
        pltpu.VMEM((K,), jnp.float32),
        pltpu.VMEM_SHARED((NS * NP,), jnp.float32),
        pltpu.SemaphoreType.DMA,
    ],
)
def _deg_kernel(src_hbm, dst_hbm, out_hbm, idxv, zbuf, ones_v, hist_sh, sem):
    c = lax.axis_index("c")
    s = lax.axis_index("s")

    @pl.when(c == 0)
    def _():
        pltpu.sync_copy(src_hbm.at[s], idxv)

    @pl.when(c == 1)
    def _():
        pltpu.sync_copy(dst_hbm.at[s], idxv)

    one16 = jnp.ones((16,), jnp.float32)
    z16 = jnp.zeros((16,), jnp.float32)

    def _fill_ones(i, _):
        ones_v[pl.ds(i * 16, 16)] = one16
        return 0
    lax.fori_loop(0, K // 16, _fill_ones, 0)

    def _fill_zero(i, _):
        zbuf[pl.ds(i * 16, 16)] = z16
        return 0
    lax.fori_loop(0, NP // 16, _fill_zero, 0)
    pltpu.sync_copy(zbuf, hist_sh.at[pl.ds(s * NP, NP)])

    off = s * NP
    def _adj(i, _):
        for j in range(K // 16):
            idxv[i, pl.ds(j * 16, 16)] = idxv[i, pl.ds(j * 16, 16)] + off
        return 0
    lax.fori_loop(0, NCHUNK, _adj, 0)

    def _fire(i, _):
        pltpu.async_copy(ones_v, hist_sh.at[idxv.at[i]], sem, add=True)
        return 0
    lax.fori_loop(0, NCHUNK, _fire, 0)

    def _drain(i, _):
        pltpu.make_async_copy(ones_v, hist_sh.at[pl.ds(0, K)], sem).wait()
        return 0
    lax.fori_loop(0, NCHUNK, _drain, 0)

    pltpu.sync_copy(hist_sh.at[pl.ds(s * NP, NP)], out_hbm.at[c * NS + s])


@functools.partial(
    pl.kernel,
    out_type=jax.ShapeDtypeStruct((NC * NP, HD), jnp.float32),
    mesh=_mesh,
    scratch_types=[
        pltpu.VMEM((EPT,), jnp.int32),
        pltpu.VMEM((K,), jnp.int32),
        pltpu.VMEM((K,), jnp.int32),
        pltpu.VMEM((K, HD), jnp.float32),
        pltpu.VMEM((K, HD), jnp.float32),
        pltpu.VMEM_SHARED((NP, HD), jnp.float32),
        pltpu.SemaphoreType.DMA,
        pltpu.SemaphoreType.DMA,
        pltpu.SemaphoreType.DMA,
        pltpu.SemaphoreType.DMA,
        pltpu.SemaphoreType.DMA,
        pltpu.SemaphoreType.DMA,
    ],
)
def _edge_kernel(h_hbm, src_hbm, dst_hbm, out_hbm,
                 srcv, db0, db1, gb0, gb1, acc,
                 dl0, dl1, gs0, gs1, sc0, sc1):
    c = lax.axis_index("c")
    s = lax.axis_index("s")
    gbufs = (gb0, gb1)
    gsems = (gs0, gs1)
    dbufs = (db0, db1)
    dlsems = (dl0, dl1)
    scsems = (sc0, sc1)

    off = c * NP
    ebase = s * EPT

    pltpu.sync_copy(src_hbm.at[pl.ds(ebase, EPT)], srcv)
    def _adj(i, _):
        srcv[pl.ds(i * 16, 16)] = srcv[pl.ds(i * 16, 16)] + off
        return 0
    lax.fori_loop(0, EPT // 16, _adj, 0)

    def _dl(i, j):
        pltpu.async_copy(dst_hbm.at[pl.ds(ebase + i * K, K)], dbufs[j], dlsems[j])

    def _dlw(j):
        pltpu.make_async_copy(dst_hbm.at[pl.ds(0, K)], dbufs[j], dlsems[j]).wait()

    def _gs(i, j):
        pltpu.async_copy(h_hbm.at[srcv.at[pl.ds(i * K, K)]], gbufs[j], gsems[j])

    def _gw(j):
        pltpu.make_async_copy(h_hbm.at[pl.ds(0, K)], gbufs[j], gsems[j]).wait()

    def _scat(j):
        pltpu.async_copy(gbufs[j], acc.at[dbufs[j]], scsems[j],
                         priority=1, add=True)
        pltpu.make_async_copy(h_hbm.at[pl.ds(0, K)], gbufs[j], scsems[j]).wait()

    z16 = jnp.zeros((16,), jnp.float32)
    def _zrow(i, _):
        for q in range(HD // 16):
            gb0[i, pl.ds(q * 16, 16)] = z16
        return 0
    lax.fori_loop(0, K, _zrow, 0)
    for q in range(RPT // K):
        pltpu.sync_copy(gb0, acc.at[pl.ds(s * RPT + q * K, K)])
    plsc.subcore_barrier()

    _dl(0, 0)
    _dl(1, 1)
    _gs(0, 0)
    def _pair(k, _):
        i0 = 2 * k
        _gw(0); _gs(i0 + 1, 1); _dlw(0); _scat(0); _dl(i0 + 2, 0)
        _gw(1); _gs(i0 + 2, 0); _dlw(1); _scat(1); _dl(i0 + 3, 1)
        return 0
    lax.fori_loop(0, NCHUNK // 2 - 1, _pair, 0)
    _gw(0); _gs(NCHUNK - 1, 1); _dlw(0); _scat(0)
    _gw(1); _dlw(1); _scat(1)
    plsc.subcore_barrier()

    pltpu.sync_copy(acc.at[pl.ds(s * RPT, RPT)],
                    out_hbm.at[pl.ds(c * NP + s * RPT, RPT)])


def _mm_pre_body(x_ref, p_ref, w_ref, out_ref, rdo_ref, rdi_ref):
    p = p_ref[...]
    so = jnp.sum(p[:, :NS], axis=1, keepdims=True)
    si = jnp.sum(p[:, NS:], axis=1, keepdims=True)
    rdo = lax.rsqrt(jnp.maximum(so, 1.0))
    rdi = lax.rsqrt(jnp.maximum(si, 1.0))
    rdo_ref[...] = rdo
    rdi_ref[...] = rdi
    xs = x_ref[...] * rdo
    h = jnp.dot(xs, w_ref[...], preferred_element_type=jnp.float32)
    out_ref[0] = h[:, :HD]
    out_ref[1] = h[:, HD:]


def _mm_mid_body(al_ref, ar_ref, rdi_ref, rdo_ref, b_ref, w_ref, out_ref):
    a = jnp.concatenate([al_ref[0], ar_ref[0]], axis=1)
    t = jnp.maximum(a * rdi_ref[...] + b_ref[...], 0.0)
    t = t * rdo_ref[...]
    h = jnp.dot(t, w_ref[...], preferred_element_type=jnp.float32)
    out_ref[0] = h[:, :HD]
    out_ref[1] = h[:, HD:]


def _fin_body(al_ref, ar_ref, rdi_ref, b_ref, out_ref):
    a = jnp.concatenate([al_ref[0], ar_ref[0]], axis=1)
    out_ref[...] = jnp.maximum(a * rdi_ref[...] + b_ref[...], 0.0)


def _mm_pre(x_pad, partials_t, W):
    return pl.pallas_call(
        _mm_pre_body,
        grid=(NBLK,),
        in_specs=[
            pl.BlockSpec((BLKR, D), lambda r: (r, 0)),
            pl.BlockSpec((BLKR, NC * NS), lambda r: (r, 0)),
            pl.BlockSpec((D, D), lambda r: (0, 0)),
        ],
        out_specs=[
            pl.BlockSpec((NC, BLKR, HD), lambda r: (0, r, 0)),
            pl.BlockSpec((BLKR, 1), lambda r: (r, 0)),
            pl.BlockSpec((BLKR, 1), lambda r: (r, 0)),
        ],
        out_shape=[
            jax.ShapeDtypeStruct((NC, NP, HD), jnp.float32),
            jax.ShapeDtypeStruct((NP, 1), jnp.float32),
            jax.ShapeDtypeStruct((NP, 1), jnp.float32),
        ],
    )(x_pad, partials_t, W)


def _mm_mid(agg, rdi, rdo, b, W):
    return pl.pallas_call(
        _mm_mid_body,
        grid=(NBLK,),
        in_specs=[
            pl.BlockSpec((1, BLKR, HD), lambda r: (0, r, 0)),
            pl.BlockSpec((1, BLKR, HD), lambda r: (1, r, 0)),
            pl.BlockSpec((BLKR, 1), lambda r: (r, 0)),
            pl.BlockSpec((BLKR, 1), lambda r: (r, 0)),
            pl.BlockSpec((1, D), lambda r: (0, 0)),
            pl.BlockSpec((D, D), lambda r: (0, 0)),
        ],
        out_specs=pl.BlockSpec((NC, BLKR, HD), lambda r: (0, r, 0)),
        out_shape=jax.ShapeDtypeStruct((NC, NP, HD), jnp.float32),
    )(agg, agg, rdi, rdo, b, W)


def _fin(agg, rdi, b):
    return pl.pallas_call(
        _fin_body,
        grid=(NBLK,),
        in_specs=[
            pl.BlockSpec((1, BLKR, HD), lambda r: (0, r, 0)),
            pl.BlockSpec((1, BLKR, HD), lambda r: (1, r, 0)),
            pl.BlockSpec((BLKR, 1), lambda r: (r, 0)),
            pl.BlockSpec((1, D), lambda r: (0, 0)),
        ],
        out_specs=pl.BlockSpec((BLKR, D), lambda r: (r, 0)),
        out_shape=jax.ShapeDtypeStruct((NP, D), jnp.float32),
    )(agg, agg, rdi, b)


def kernel(features, edge_index, W1, b1, W2, b2):
    pad = (jnp.arange(EPAD - E, dtype=jnp.int32) % (NP - N)) + N
    src = jnp.concatenate([edge_index[0], pad])
    dst = jnp.concatenate([edge_index[1], pad])
    src3 = src.reshape(NS, NCHUNK, K)
    dst3 = dst.reshape(NS, NCHUNK, K)

    x_pad = jnp.pad(features, ((0, NP - N), (0, 0)))

    partials = _deg_kernel(src3, dst3)

    b1r = b1.reshape(1, D)
    b2r = b2.reshape(1, D)

    h, rdo, rdi = _mm_pre(x_pad, partials.T, W1)
    agg = _edge_kernel(h.reshape(NC * NP, HD), src, dst).reshape(NC, NP, HD)
    g = _mm_mid(agg, rdi, rdo, b1r, W2)
    agg2 = _edge_kernel(g.reshape(NC * NP, HD), src, dst).reshape(NC, NP, HD)
    out = _fin(agg2, rdi, b2r)
    return out[:N]
